# baseline (speedup 1.0000x reference)
import functools

import jax
import jax.numpy as jnp
from jax import lax
from jax.experimental import pallas as pl
from jax.experimental.pallas import tpu as pltpu
from jax.experimental.pallas import tpu_sc as plsc


def _sc_gather(emb, idx_flat):
    info = plsc.get_sparse_core_info()
    nw = info.num_cores * info.num_subcores
    b = idx_flat.shape[0]
    d = emb.shape[1]
    b_per_w = b // nw
    mesh = plsc.VectorSubcoreMesh(core_axis_name="c", subcore_axis_name="s")

    @functools.partial(
        pl.kernel,
        mesh=mesh,
        compiler_params=pltpu.CompilerParams(use_tc_tiling_on_sc=False),
        out_type=jax.ShapeDtypeStruct((b, d), jnp.float32),
        scratch_types=[
            pltpu.VMEM((b_per_w,), jnp.int32),
            pltpu.VMEM((b_per_w, d), jnp.float32),
            pltpu.SemaphoreType.DMA,
        ],
    )
    def gather_kernel(table_hbm, idx_hbm, out_hbm, idx_v, rows_v, sem):
        wid = lax.axis_index("s") * info.num_cores + lax.axis_index("c")
        base = wid * b_per_w
        pltpu.sync_copy(idx_hbm.at[pl.ds(base, b_per_w)], idx_v)
        pltpu.async_copy(table_hbm.at[idx_v], rows_v, sem).wait()
        pltpu.sync_copy(rows_v, out_hbm.at[pl.ds(base, b_per_w)])

    return gather_kernel(emb, idx_flat)


def _mlp_body(g_ref, w1_ref, b1_ref, w2_ref, b2_ref, out_ref, h_ref):
    @pl.when(pl.program_id(0) == 0)
    def _():
        h = jnp.dot(g_ref[...], w1_ref[...], preferred_element_type=jnp.float32)
        h_ref[...] = jnp.maximum(h + b1_ref[...], 0.0)

    out_ref[...] = (
        jnp.dot(h_ref[...], w2_ref[...], preferred_element_type=jnp.float32)
        + b2_ref[...]
    )


def _mlp(g, W1, b1, W2, b2, tn=2048):
    batch, feat = g.shape
    hidden = W1.shape[1]
    vocab = W2.shape[1]
    num_tiles = pl.cdiv(vocab, tn)
    b1r = b1.reshape(1, hidden)
    b2r = b2.reshape(1, vocab)
    return pl.pallas_call(
        _mlp_body,
        grid=(num_tiles,),
        in_specs=[
            pl.BlockSpec((batch, feat), lambda j: (0, 0)),
            pl.BlockSpec((feat, hidden), lambda j: (0, 0)),
            pl.BlockSpec((1, hidden), lambda j: (0, 0)),
            pl.BlockSpec((hidden, tn), lambda j: (0, j)),
            pl.BlockSpec((1, tn), lambda j: (0, j)),
        ],
        out_specs=pl.BlockSpec((batch, tn), lambda j: (0, j)),
        out_shape=jax.ShapeDtypeStruct((batch, vocab), jnp.float32),
        scratch_shapes=[pltpu.VMEM((batch, hidden), jnp.float32)],
    )(g, W1, b1r, W2, b2r)


def kernel(x, emb, W1, b1, W2, b2):
    batch, block_size = x.shape
    emb_dim = emb.shape[1]
    idx_flat = x.reshape(-1).astype(jnp.int32)
    rows = _sc_gather(emb, idx_flat)
    g = rows.reshape(batch, block_size * emb_dim)
    return _mlp(g, W1, b1, W2, b2)


# speedup vs baseline: 1.0007x; 1.0007x over previous
import functools

import jax
import jax.numpy as jnp
from jax import lax
from jax.experimental import pallas as pl
from jax.experimental.pallas import tpu as pltpu
from jax.experimental.pallas import tpu_sc as plsc


def _sc_gather(emb, idx_flat):
    info = plsc.get_sparse_core_info()
    nw = info.num_cores * info.num_subcores
    b = idx_flat.shape[0]
    d = emb.shape[1]
    b_per_w = b // nw
    mesh = plsc.VectorSubcoreMesh(core_axis_name="c", subcore_axis_name="s")

    @functools.partial(
        pl.kernel,
        mesh=mesh,
        compiler_params=pltpu.CompilerParams(use_tc_tiling_on_sc=False),
        out_type=jax.ShapeDtypeStruct((b, d), jnp.float32),
        scratch_types=[
            pltpu.VMEM((b_per_w,), jnp.int32),
            pltpu.VMEM((b_per_w, d), jnp.float32),
            pltpu.SemaphoreType.DMA,
        ],
    )
    def gather_kernel(table_hbm, idx_hbm, out_hbm, idx_v, rows_v, sem):
        wid = lax.axis_index("s") * info.num_cores + lax.axis_index("c")
        base = wid * b_per_w
        pltpu.sync_copy(idx_hbm.at[pl.ds(base, b_per_w)], idx_v)
        pltpu.async_copy(table_hbm.at[idx_v], rows_v, sem).wait()
        pltpu.sync_copy(rows_v, out_hbm.at[pl.ds(base, b_per_w)])

    return gather_kernel(emb, idx_flat)


def _mlp_body(g_ref, w1_ref, b1_ref, w2_ref, b2_ref, out_ref, h_ref):
    @pl.when(pl.program_id(0) == 0)
    def _():
        h = jnp.dot(g_ref[...], w1_ref[...], preferred_element_type=jnp.float32)
        h_ref[...] = jnp.maximum(h + b1_ref[...], 0.0).astype(jnp.bfloat16)

    out_ref[...] = (
        jnp.dot(
            h_ref[...],
            w2_ref[...].astype(jnp.bfloat16),
            preferred_element_type=jnp.float32,
        )
        + b2_ref[...]
    )


def _mlp(g, W1, b1, W2, b2, tn=2048):
    batch, feat = g.shape
    hidden = W1.shape[1]
    vocab = W2.shape[1]
    num_tiles = pl.cdiv(vocab, tn)
    b1r = b1.reshape(1, hidden)
    b2r = b2.reshape(1, vocab)
    return pl.pallas_call(
        _mlp_body,
        grid=(num_tiles,),
        in_specs=[
            pl.BlockSpec((batch, feat), lambda j: (0, 0)),
            pl.BlockSpec((feat, hidden), lambda j: (0, 0)),
            pl.BlockSpec((1, hidden), lambda j: (0, 0)),
            pl.BlockSpec((hidden, tn), lambda j: (0, j)),
            pl.BlockSpec((1, tn), lambda j: (0, j)),
        ],
        out_specs=pl.BlockSpec((batch, tn), lambda j: (0, j)),
        out_shape=jax.ShapeDtypeStruct((batch, vocab), jnp.float32),
        scratch_shapes=[pltpu.VMEM((batch, hidden), jnp.bfloat16)],
    )(g, W1, b1r, W2, b2r)


def kernel(x, emb, W1, b1, W2, b2):
    batch, block_size = x.shape
    emb_dim = emb.shape[1]
    idx_flat = x.reshape(-1).astype(jnp.int32)
    rows = _sc_gather(emb, idx_flat)
    g = rows.reshape(batch, block_size * emb_dim)
    return _mlp(g, W1, b1, W2, b2)
